# baseline (speedup 1.0000x reference)
import functools

import jax
import jax.numpy as jnp
from jax import lax
from jax.experimental import pallas as pl
from jax.experimental.pallas import tpu as pltpu
from jax.experimental.pallas import tpu_sc as plsc

NUM_GRAPHS = 512
CHUNK = 128
GROUP = 5
BLK = 4096
NC, NS = 2, 16


def _gnn_matmul(x, ids3d, w, bvec, n_pad):
    n, d = x.shape

    def body(x_ref, ids_ref, w_ref, b_ref, o_ref, cnt_ref):
        i = pl.program_id(0)
        acc = lax.dot_general(
            x_ref[...].astype(jnp.bfloat16), w_ref[...].astype(jnp.bfloat16),
            (((1,), (0,)), ((), ())),
            preferred_element_type=jnp.float32)
        o_ref[...] = jnp.maximum(acc + b_ref[...], 0.0)

        gids = lax.broadcasted_iota(jnp.int32, (NUM_GRAPHS, BLK), 0)
        onehot_t = jnp.where(ids_ref[0] == gids, 1.0, 0.0)
        ones_col = jnp.ones((BLK, 1), jnp.float32)
        part = lax.dot_general(
            onehot_t, ones_col, (((1,), (0,)), ((), ())),
            preferred_element_type=jnp.float32)

        @pl.when(i == 0)
        def _init():
            cnt_ref[...] = jnp.zeros_like(cnt_ref)

        cnt_ref[...] += part

    return pl.pallas_call(
        body,
        grid=(n_pad // BLK,),
        in_specs=[
            pl.BlockSpec((BLK, d), lambda i: (i, 0)),
            pl.BlockSpec((1, 1, BLK), lambda i: (i, 0, 0)),
            pl.BlockSpec((d, d), lambda i: (0, 0)),
            pl.BlockSpec((1, d), lambda i: (0, 0)),
        ],
        out_specs=[
            pl.BlockSpec((BLK, d), lambda i: (i, 0)),
            pl.BlockSpec((NUM_GRAPHS, 1), lambda i: (0, 0)),
        ],
        out_shape=[
            jax.ShapeDtypeStruct((n_pad, d), jnp.float32),
            jax.ShapeDtypeStruct((NUM_GRAPHS, 1), jnp.float32),
        ],
    )(x, ids3d, w, bvec.reshape(1, d))


def _sc_segment_sum(node_rep, ids2d, zeros_sum):
    d = node_rep.shape[1]
    n_groups = ids2d.shape[0]
    nw = NC * NS
    mesh = plsc.VectorSubcoreMesh(core_axis_name="c", subcore_axis_name="s")

    @functools.partial(
        pl.kernel,
        out_type=jax.ShapeDtypeStruct((NC, NUM_GRAPHS, d), jnp.float32),
        mesh=mesh,
        scratch_types=[
            pltpu.VMEM((GROUP, CHUNK), jnp.int32),
            pltpu.VMEM((GROUP * CHUNK, d), jnp.float32),
            pltpu.VMEM_SHARED((NUM_GRAPHS + 1, d), jnp.float32),
        ],
    )
    def k(rep_hbm, ids_hbm, z_sum_hbm, out_sum, idx_v, rows_v, acc_sh):
        c = lax.axis_index("c")
        s = lax.axis_index("s")
        wid = s * NC + c

        @pl.when(s == 0)
        def _zero():
            pltpu.sync_copy(z_sum_hbm, acc_sh)

        plsc.subcore_barrier()

        n_mine = (n_groups - wid + nw - 1) // nw

        def body(g, carry):
            grp = wid + g * nw
            pltpu.sync_copy(ids_hbm.at[grp], idx_v)
            pltpu.sync_copy(
                rep_hbm.at[pl.ds(grp * GROUP * CHUNK, GROUP * CHUNK)], rows_v)
            for j in range(GROUP):
                pltpu.sync_copy(rows_v.at[pl.ds(j * CHUNK, CHUNK)],
                                acc_sh.at[idx_v.at[j]], add=True)
            return carry

        lax.fori_loop(0, n_mine, body, 0)
        plsc.subcore_barrier()

        r0 = s * (NUM_GRAPHS // NS)
        pltpu.sync_copy(acc_sh.at[pl.ds(r0, NUM_GRAPHS // NS)],
                        out_sum.at[c, pl.ds(r0, NUM_GRAPHS // NS)])

    return k(node_rep, ids2d, zeros_sum)


def _head(psum, cnt_col, w, bvec):
    t = w.shape[1]

    def body(ps_ref, cnt_ref, w_ref, b_ref, o_ref):
        seg = ps_ref[0] + ps_ref[1]
        rep = seg / jnp.maximum(cnt_ref[...], 1.0)
        o_ref[...] = (
            lax.dot_general(rep, w_ref[...], (((1,), (0,)), ((), ())),
                            precision=lax.Precision.HIGHEST,
                            preferred_element_type=jnp.float32)
            + b_ref[...]
        )

    return pl.pallas_call(
        body,
        out_shape=jax.ShapeDtypeStruct((NUM_GRAPHS, t), jnp.float32),
    )(psum, cnt_col, w, bvec.reshape(1, t))


def kernel(x, batch, W_gnn, b_gnn, W, b):
    n, d = x.shape
    n_pad = ((n + BLK - 1) // BLK) * BLK

    ids = jnp.concatenate(
        [batch.astype(jnp.int32),
         jnp.full((n_pad - n,), NUM_GRAPHS, jnp.int32)])
    ids2d = ids.reshape(-1, GROUP, CHUNK)
    ids3d = ids.reshape(-1, 1, BLK)
    zeros_sum = jnp.zeros((NUM_GRAPHS + 1, d), jnp.float32)

    node_rep, cnt_col = _gnn_matmul(x, ids3d, W_gnn, b_gnn, n_pad)
    psum = _sc_segment_sum(node_rep, ids2d, zeros_sum)
    return _head(psum, cnt_col, W, b)


# speedup vs baseline: 1.0852x; 1.0852x over previous
import functools

import jax
import jax.numpy as jnp
from jax import lax
from jax.experimental import pallas as pl
from jax.experimental.pallas import tpu as pltpu
from jax.experimental.pallas import tpu_sc as plsc

NUM_GRAPHS = 512
HBINS = 544
CHUNK = 128
GROUP = 5
BLK = 4096
NC, NS = 2, 16


def _gnn_matmul(x, w, bvec, n_pad):
    n, d = x.shape

    def body(x_ref, w_ref, b_ref, o_ref):
        acc = lax.dot_general(
            x_ref[...].astype(jnp.bfloat16), w_ref[...].astype(jnp.bfloat16),
            (((1,), (0,)), ((), ())),
            preferred_element_type=jnp.float32)
        o_ref[...] = jnp.maximum(acc + b_ref[...], 0.0)

    return pl.pallas_call(
        body,
        grid=(n_pad // BLK,),
        in_specs=[
            pl.BlockSpec((BLK, d), lambda i: (i, 0)),
            pl.BlockSpec((d, d), lambda i: (0, 0)),
            pl.BlockSpec((1, d), lambda i: (0, 0)),
        ],
        out_specs=pl.BlockSpec((BLK, d), lambda i: (i, 0)),
        out_shape=jax.ShapeDtypeStruct((n_pad, d), jnp.float32),
    )(x, w, bvec.reshape(1, d))


def _sc_segment_sum(node_rep, ids3d, zeros_sum):
    d = node_rep.shape[1]
    n_groups = ids3d.shape[0]
    nw = NC * NS
    mesh = plsc.VectorSubcoreMesh(core_axis_name="c", subcore_axis_name="s")

    @functools.partial(
        pl.kernel,
        out_type=[
            jax.ShapeDtypeStruct((NC, NUM_GRAPHS, d), jnp.float32),
            jax.ShapeDtypeStruct((NC, NS, HBINS), jnp.float32),
        ],
        mesh=mesh,
        scratch_types=[
            pltpu.VMEM((GROUP, CHUNK), jnp.int32),
            pltpu.VMEM((GROUP * CHUNK, d), jnp.float32),
            pltpu.VMEM((HBINS,), jnp.float32),
            pltpu.VMEM_SHARED((NUM_GRAPHS + 1, d), jnp.float32),
        ],
        compiler_params=pltpu.CompilerParams(needs_layout_passes=False),
    )
    def k(rep_hbm, ids_hbm, z_sum_hbm, out_sum, out_cnt,
          idx_v, rows_v, hist_v, acc_sh):
        c = lax.axis_index("c")
        s = lax.axis_index("s")
        wid = s * NC + c

        @pl.when(s == 0)
        def _zero():
            pltpu.sync_copy(z_sum_hbm, acc_sh)

        zeros16 = jnp.zeros((16,), jnp.float32)
        for kk in range(HBINS // 16):
            hist_v[pl.ds(kk * 16, 16)] = zeros16

        plsc.subcore_barrier()

        n_mine = (n_groups - wid + nw - 1) // nw
        ones16 = jnp.ones((16,), jnp.float32)

        def body(g, carry):
            grp = wid + g * nw
            pltpu.sync_copy(ids_hbm.at[grp], idx_v)
            pltpu.sync_copy(
                rep_hbm.at[pl.ds(grp * GROUP * CHUNK, GROUP * CHUNK)], rows_v)
            for j in range(GROUP):
                pltpu.sync_copy(rows_v.at[pl.ds(j * CHUNK, CHUNK)],
                                acc_sh.at[idx_v.at[j]], add=True)
                for kk in range(CHUNK // 16):
                    vec = idx_v[j, pl.ds(kk * 16, 16)]
                    plsc.addupdate_scatter(hist_v, [vec], ones16)
            return carry

        lax.fori_loop(0, n_mine, body, 0)
        pltpu.sync_copy(hist_v, out_cnt.at[c, s])
        plsc.subcore_barrier()

        r0 = s * (NUM_GRAPHS // NS)
        pltpu.sync_copy(acc_sh.at[pl.ds(r0, NUM_GRAPHS // NS)],
                        out_sum.at[c, pl.ds(r0, NUM_GRAPHS // NS)])

    return k(node_rep, ids3d, zeros_sum)


def _head(psum, hists, w, bvec):
    t = w.shape[1]

    def body(ps_ref, h_ref, w_ref, b_ref, o_ref):
        seg = ps_ref[0] + ps_ref[1]
        cnt_row = jnp.sum(h_ref[...].reshape(NC * NS, HBINS), axis=0,
                          keepdims=True)[:, 0:NUM_GRAPHS]
        ident = jnp.where(
            lax.broadcasted_iota(jnp.int32, (NUM_GRAPHS, NUM_GRAPHS), 0)
            == lax.broadcasted_iota(jnp.int32, (NUM_GRAPHS, NUM_GRAPHS), 1),
            1.0, 0.0)
        cnt = lax.dot_general(ident, cnt_row, (((1,), (1,)), ((), ())),
                              precision=lax.Precision.HIGHEST,
                              preferred_element_type=jnp.float32)
        rep = seg / jnp.maximum(cnt, 1.0)
        o_ref[...] = (
            lax.dot_general(rep, w_ref[...], (((1,), (0,)), ((), ())),
                            precision=lax.Precision.HIGHEST,
                            preferred_element_type=jnp.float32)
            + b_ref[...]
        )

    return pl.pallas_call(
        body,
        out_shape=jax.ShapeDtypeStruct((NUM_GRAPHS, t), jnp.float32),
    )(psum, hists, w, bvec.reshape(1, t))


def kernel(x, batch, W_gnn, b_gnn, W, b):
    n, d = x.shape
    n_pad = ((n + BLK - 1) // BLK) * BLK

    ids = jnp.concatenate(
        [batch.astype(jnp.int32),
         jnp.full((n_pad - n,), NUM_GRAPHS, jnp.int32)])
    ids3d = ids.reshape(-1, GROUP, CHUNK)
    zeros_sum = jnp.zeros((NUM_GRAPHS + 1, d), jnp.float32)

    node_rep = _gnn_matmul(x, W_gnn, b_gnn, n_pad)
    psum, hists = _sc_segment_sum(node_rep, ids3d, zeros_sum)
    return _head(psum, hists, W, b)


# speedup vs baseline: 1.1090x; 1.0220x over previous
import functools

import jax
import jax.numpy as jnp
from jax import lax
from jax.experimental import pallas as pl
from jax.experimental.pallas import tpu as pltpu
from jax.experimental.pallas import tpu_sc as plsc

NUM_GRAPHS = 512
HBINS = 544
CHUNK = 128
GROUP = 5
NGRP = 5
BLK = 4096
NC, NS = 2, 16


def _gnn_matmul(x, w, bvec, n_pad):
    n, d = x.shape

    def body(x_ref, w_ref, b_ref, o_ref):
        acc = lax.dot_general(
            x_ref[...].astype(jnp.bfloat16), w_ref[...].astype(jnp.bfloat16),
            (((1,), (0,)), ((), ())),
            preferred_element_type=jnp.float32)
        o_ref[...] = jnp.maximum(acc + b_ref[...], 0.0)

    return pl.pallas_call(
        body,
        grid=(n_pad // BLK,),
        in_specs=[
            pl.BlockSpec((BLK, d), lambda i: (i, 0)),
            pl.BlockSpec((d, d), lambda i: (0, 0)),
            pl.BlockSpec((1, d), lambda i: (0, 0)),
        ],
        out_specs=pl.BlockSpec((BLK, d), lambda i: (i, 0)),
        out_shape=jax.ShapeDtypeStruct((n_pad, d), jnp.float32),
    )(x, w, bvec.reshape(1, d))


def _sc_segment_sum(node_rep, ids_w, zeros_sum):
    d = node_rep.shape[1]
    nw = NC * NS
    n_mine = ids_w.shape[1] // GROUP
    mesh = plsc.VectorSubcoreMesh(core_axis_name="c", subcore_axis_name="s")

    @functools.partial(
        pl.kernel,
        out_type=[
            jax.ShapeDtypeStruct((NC, NUM_GRAPHS, d), jnp.float32),
            jax.ShapeDtypeStruct((NC, NS, HBINS), jnp.float32),
        ],
        mesh=mesh,
        scratch_types=[
            pltpu.VMEM((NGRP * GROUP, CHUNK), jnp.int32),
            pltpu.VMEM((GROUP * CHUNK, d), jnp.float32),
            pltpu.VMEM((HBINS,), jnp.float32),
            pltpu.VMEM_SHARED((NUM_GRAPHS + 1, d), jnp.float32),
        ],
        compiler_params=pltpu.CompilerParams(needs_layout_passes=False),
    )
    def k(rep_hbm, ids_hbm, z_sum_hbm, out_sum, out_cnt,
          idx_v, rows_v, hist_v, acc_sh):
        c = lax.axis_index("c")
        s = lax.axis_index("s")
        wid = s * NC + c

        @pl.when(s == 0)
        def _zero():
            pltpu.sync_copy(z_sum_hbm, acc_sh)

        pltpu.sync_copy(ids_hbm.at[wid], idx_v)

        zeros16 = jnp.zeros((16,), jnp.float32)
        for kk in range(HBINS // 16):
            hist_v[pl.ds(kk * 16, 16)] = zeros16

        plsc.subcore_barrier()

        ones16 = jnp.ones((16,), jnp.float32)

        for g in range(n_mine):
            grp = wid + g * nw
            pltpu.sync_copy(
                rep_hbm.at[pl.ds(grp * GROUP * CHUNK, GROUP * CHUNK)], rows_v)
            for j in range(GROUP):
                r = g * GROUP + j
                pltpu.sync_copy(rows_v.at[pl.ds(j * CHUNK, CHUNK)],
                                acc_sh.at[idx_v.at[r]], add=True)
                for kk in range(CHUNK // 16):
                    vec = idx_v[r, pl.ds(kk * 16, 16)]
                    plsc.addupdate_scatter(hist_v, [vec], ones16)
        pltpu.sync_copy(hist_v, out_cnt.at[c, s])
        plsc.subcore_barrier()

        r0 = s * (NUM_GRAPHS // NS)
        pltpu.sync_copy(acc_sh.at[pl.ds(r0, NUM_GRAPHS // NS)],
                        out_sum.at[c, pl.ds(r0, NUM_GRAPHS // NS)])

    return k(node_rep, ids_w, zeros_sum)


def _head(psum, hists, w, bvec):
    t = w.shape[1]

    def body(ps_ref, h_ref, w_ref, b_ref, o_ref):
        seg = ps_ref[0] + ps_ref[1]
        cnt_row = jnp.sum(h_ref[...].reshape(NC * NS, HBINS), axis=0,
                          keepdims=True)[:, 0:NUM_GRAPHS]
        ident = jnp.where(
            lax.broadcasted_iota(jnp.int32, (NUM_GRAPHS, NUM_GRAPHS), 0)
            == lax.broadcasted_iota(jnp.int32, (NUM_GRAPHS, NUM_GRAPHS), 1),
            1.0, 0.0)
        cnt = lax.dot_general(ident, cnt_row, (((1,), (1,)), ((), ())),
                              precision=lax.Precision.HIGHEST,
                              preferred_element_type=jnp.float32)
        rep = seg / jnp.maximum(cnt, 1.0)
        o_ref[...] = (
            lax.dot_general(rep, w_ref[...], (((1,), (0,)), ((), ())),
                            precision=lax.Precision.HIGHEST,
                            preferred_element_type=jnp.float32)
            + b_ref[...]
        )

    return pl.pallas_call(
        body,
        out_shape=jax.ShapeDtypeStruct((NUM_GRAPHS, t), jnp.float32),
    )(psum, hists, w, bvec.reshape(1, t))


def kernel(x, batch, W_gnn, b_gnn, W, b):
    n, d = x.shape
    n_pad = ((n + BLK - 1) // BLK) * BLK

    ids = jnp.concatenate(
        [batch.astype(jnp.int32),
         jnp.full((n_pad - n,), NUM_GRAPHS, jnp.int32)])
    nw = NC * NS
    ids_w = (ids.reshape(NGRP, nw, GROUP, CHUNK)
             .transpose(1, 0, 2, 3)
             .reshape(nw, NGRP * GROUP, CHUNK))
    zeros_sum = jnp.zeros((NUM_GRAPHS + 1, d), jnp.float32)

    node_rep = _gnn_matmul(x, W_gnn, b_gnn, n_pad)
    psum, hists = _sc_segment_sum(node_rep, ids_w, zeros_sum)
    return _head(psum, hists, W, b)
